# Initial kernel scaffold; baseline (speedup 1.0000x reference)
import jax
import jax.numpy as jnp
from jax.experimental import pallas as pl


def kernel(x, edge_index, W_l, W_r, b):
    raise NotImplementedError("write your pallas kernel here")


# speedup vs baseline: 1.5519x; 1.5519x over previous
import functools

import jax
import jax.numpy as jnp
from jax import lax
from jax.experimental import pallas as pl
from jax.experimental.pallas import tpu as pltpu
from jax.experimental.pallas import tpu_sc as plsc

N, E, D = 10000, 160000, 256

NC, NS, L = 2, 16, 16
NW = NC * NS
HD = D // 2
QHALF = 5000
QMAX = 2504
ACC_ROWS = 2528
TRASH = QMAX
CROWS = 20
OUT_ROWS = 5008
CHUNK = 128
NCHUNKS = E // CHUNK
ZCHUNK = 32
WB = 128
WB_FULL = 19
WB_TAIL = 72


def _make_sc_aggregate(base):
  mesh = plsc.VectorSubcoreMesh(
      core_axis_name="c", subcore_axis_name="s", num_cores=NC, num_subcores=NS)

  @functools.partial(
      pl.kernel,
      out_type=(
          jax.ShapeDtypeStruct((OUT_ROWS, HD), jnp.float32),
          jax.ShapeDtypeStruct((OUT_ROWS, HD), jnp.float32),
          jax.ShapeDtypeStruct((OUT_ROWS, HD), jnp.float32),
      ),
      mesh=mesh,
      compiler_params=pltpu.CompilerParams(needs_layout_passes=False),
      scratch_types=[
          pltpu.VMEM((CHUNK,), jnp.int32),
          pltpu.VMEM((CHUNK,), jnp.int32),
          pltpu.VMEM((CHUNK,), jnp.int32),
          pltpu.VMEM((CHUNK,), jnp.int32),
          pltpu.VMEM((CHUNK, HD), jnp.float32),
          pltpu.VMEM((CHUNK, HD), jnp.float32),
          pltpu.VMEM((CHUNK, HD), jnp.float32),
          pltpu.VMEM((CROWS, HD), jnp.float32),
          pltpu.VMEM_SHARED((ACC_ROWS, HD), jnp.float32),
          pltpu.VMEM_SHARED((ACC_ROWS, HD), jnp.float32),
          pltpu.VMEM_SHARED((CROWS, HD), jnp.float32),
          pltpu.SemaphoreType.DMA,
          pltpu.SemaphoreType.DMA,
      ],
  )
  def agg(xa_hbm, xb_hbm, src_hbm, dst_hbm, suma_hbm, sumb_hbm, counts_hbm,
          src_v, dst_v, ld_v, cr_v, rowa, rowb, onesbuf, ccopy,
          acca, accb, cacc, sema, semb):
    c = lax.axis_index("c")
    s = lax.axis_index("s")
    wid = s * NC + c
    out0 = c * QMAX
    lo = base + out0
    hi = jnp.minimum(lo + QMAX, base + QHALF)
    lane = jnp.arange(L, dtype=jnp.int32)
    col0 = jnp.zeros((L,), jnp.int32)
    ones16 = jnp.ones((L,), jnp.float32)
    zeros16 = jnp.zeros((L,), jnp.float32)

    def zero_row(i, _):
      for j in range(HD // L):
        rowa[i, pl.ds(j * L, L)] = zeros16
        onesbuf[i, pl.ds(j * L, L)] = zeros16
      return 0
    lax.fori_loop(0, CHUNK, zero_row, 0)

    nzch = ACC_ROWS // ZCHUNK
    nz = (nzch - s + NS - 1) // NS

    def zero_acc(i, _):
      r0 = (s + i * NS) * ZCHUNK
      pltpu.sync_copy(rowa.at[pl.ds(0, ZCHUNK)], acca.at[pl.ds(r0, ZCHUNK)])
      pltpu.sync_copy(rowa.at[pl.ds(0, ZCHUNK)], accb.at[pl.ds(r0, ZCHUNK)])
      return 0
    lax.fori_loop(0, nz, zero_acc, 0)

    @pl.when(s == 0)
    def _zero_cacc():
      pltpu.sync_copy(onesbuf.at[pl.ds(0, CROWS)], cacc)

    plsc.subcore_barrier()

    nj = (NCHUNKS - s + NS - 1) // NS

    def body(i, _):
      off = (s + i * NS) * CHUNK
      pltpu.sync_copy(src_hbm.at[pl.ds(off, CHUNK)], src_v)
      pltpu.sync_copy(dst_hbm.at[pl.ds(off, CHUNK)], dst_v)
      cpa = pltpu.async_copy(xa_hbm.at[src_v], rowa, sema)
      cpb = pltpu.async_copy(xb_hbm.at[src_v], rowb, semb)
      for t in range(CHUNK // L):
        dv = dst_v[pl.ds(t * L, L)]
        in_range = (dv >= lo) & (dv < hi)
        ld = jnp.where(in_range, dv - lo, TRASH)
        ld_v[pl.ds(t * L, L)] = ld
        cr_v[pl.ds(t * L, L)] = lax.shift_right_logical(ld, 7)
        plsc.store_scatter(onesbuf, [lane + t * L, lax.bitwise_and(ld, 127)],
                           ones16)
      cpa.wait()
      pltpu.sync_copy(rowa, acca.at[ld_v], add=True)
      cpb.wait()
      pltpu.sync_copy(rowb, accb.at[ld_v], add=True)
      pltpu.sync_copy(onesbuf, cacc.at[cr_v], add=True)
      for t in range(CHUNK // L):
        ld = ld_v[pl.ds(t * L, L)]
        plsc.store_scatter(onesbuf, [lane + t * L, lax.bitwise_and(ld, 127)],
                           zeros16)
      return 0
    lax.fori_loop(0, nj, body, 0)

    plsc.subcore_barrier()

    pltpu.sync_copy(cacc, ccopy)

    def emit_counts(n0, ngroups):
      for t in range(ngroups):
        vals = ccopy[n0 // HD, pl.ds(t * L, L)]
        plsc.store_scatter(rowa, [lane + t * L, col0], vals)

    nwb = (WB_FULL - s + NS - 1) // NS

    def wb_body(i, _):
      r0 = (s + i * NS) * WB
      pltpu.sync_copy(acca.at[pl.ds(r0, WB)], rowa)
      pltpu.sync_copy(rowa, suma_hbm.at[pl.ds(out0 + r0, WB)])
      pltpu.sync_copy(accb.at[pl.ds(r0, WB)], rowb)
      pltpu.sync_copy(rowb, sumb_hbm.at[pl.ds(out0 + r0, WB)])
      return 0
    lax.fori_loop(0, nwb, wb_body, 0)

    def wbc_body(i, _):
      r0 = (s + i * NS) * WB
      emit_counts(r0, WB // L)
      pltpu.sync_copy(rowa, counts_hbm.at[pl.ds(out0 + r0, WB)])
      return 0
    lax.fori_loop(0, nwb, wbc_body, 0)

    @pl.when(s == NS - 1)
    def _tail():
      r0 = WB_FULL * WB
      pltpu.sync_copy(acca.at[pl.ds(r0, WB_TAIL)], rowa.at[pl.ds(0, WB_TAIL)])
      pltpu.sync_copy(rowa.at[pl.ds(0, WB_TAIL)],
                      suma_hbm.at[pl.ds(out0 + r0, WB_TAIL)])
      pltpu.sync_copy(accb.at[pl.ds(r0, WB_TAIL)], rowb.at[pl.ds(0, WB_TAIL)])
      pltpu.sync_copy(rowb.at[pl.ds(0, WB_TAIL)],
                      sumb_hbm.at[pl.ds(out0 + r0, WB_TAIL)])
      emit_counts(r0, (WB_TAIL + L - 1) // L)
      pltpu.sync_copy(rowa.at[pl.ds(0, WB_TAIL)],
                      counts_hbm.at[pl.ds(out0 + r0, WB_TAIL)])

  return agg


BLK = 400


def _tc_body(x_ref, suma_ref, sumb_ref, counts_ref, wl_ref, wr_ref, b_ref,
             o_ref):
  cnt = jnp.maximum(counts_ref[:, 0:1], 1.0)
  meana = suma_ref[...] / cnt
  meanb = sumb_ref[...] / cnt
  o_ref[...] = (
      jnp.dot(meana, wl_ref[0:HD, :], preferred_element_type=jnp.float32,
              precision=lax.Precision.HIGHEST)
      + jnp.dot(meanb, wl_ref[HD:D, :], preferred_element_type=jnp.float32,
                precision=lax.Precision.HIGHEST)
      + jnp.dot(x_ref[...], wr_ref[...], preferred_element_type=jnp.float32,
                precision=lax.Precision.HIGHEST)
      + b_ref[...])


def _tc_update(x, suma, sumb, counts, W_l, W_r, b):
  return pl.pallas_call(
      _tc_body,
      grid=(N // BLK,),
      in_specs=[
          pl.BlockSpec((BLK, D), lambda i: (i, 0)),
          pl.BlockSpec((BLK, HD), lambda i: (i, 0)),
          pl.BlockSpec((BLK, HD), lambda i: (i, 0)),
          pl.BlockSpec((BLK, HD), lambda i: (i, 0)),
          pl.BlockSpec((D, D), lambda i: (0, 0)),
          pl.BlockSpec((D, D), lambda i: (0, 0)),
          pl.BlockSpec((1, D), lambda i: (0, 0)),
      ],
      out_specs=pl.BlockSpec((BLK, D), lambda i: (i, 0)),
      out_shape=jax.ShapeDtypeStruct((N, D), jnp.float32),
  )(x, suma, sumb, counts, W_l, W_r, b.reshape(1, D))


@jax.jit
def kernel(x, edge_index, W_l, W_r, b):
  src = edge_index[0]
  dst = edge_index[1]
  xa = x[:, :HD]
  xb = x[:, HD:]
  suma0, sumb0, counts0 = _make_sc_aggregate(0)(xa, xb, src, dst)
  suma1, sumb1, counts1 = _make_sc_aggregate(QHALF)(xa, xb, src, dst)
  suma = jnp.concatenate([suma0[:QHALF], suma1[:QHALF]])
  sumb = jnp.concatenate([sumb0[:QHALF], sumb1[:QHALF]])
  counts = jnp.concatenate([counts0[:QHALF], counts1[:QHALF]])
  return _tc_update(x, suma, sumb, counts, W_l, W_r, b)


# speedup vs baseline: 1.7404x; 1.1215x over previous
import functools

import jax
import jax.numpy as jnp
from jax import lax
from jax.experimental import pallas as pl
from jax.experimental.pallas import tpu as pltpu
from jax.experimental.pallas import tpu_sc as plsc

N, E, D = 10000, 160000, 256

NC, NS, L = 2, 16, 16
NW = NC * NS
HD = D // 2
QHALF = 5000
QMAX = 2504
ACC_ROWS = 2528
TRASH = QMAX
CROWS = 20
OUT_ROWS = 5008
CHUNK = 128
NCHUNKS = E // CHUNK
ZCHUNK = 32
WB = 128
WB_FULL = 19
WB_TAIL = 72


def _make_sc_aggregate(base):
  mesh = plsc.VectorSubcoreMesh(
      core_axis_name="c", subcore_axis_name="s", num_cores=NC, num_subcores=NS)

  @functools.partial(
      pl.kernel,
      out_type=(
          jax.ShapeDtypeStruct((OUT_ROWS, HD), jnp.float32),
          jax.ShapeDtypeStruct((OUT_ROWS, HD), jnp.float32),
          jax.ShapeDtypeStruct((OUT_ROWS, HD), jnp.float32),
      ),
      mesh=mesh,
      compiler_params=pltpu.CompilerParams(needs_layout_passes=False),
      scratch_types=[
          pltpu.VMEM((CHUNK,), jnp.int32),
          pltpu.VMEM((CHUNK,), jnp.int32),
          pltpu.VMEM((CHUNK,), jnp.int32),
          pltpu.VMEM((CHUNK,), jnp.int32),
          pltpu.VMEM((CHUNK,), jnp.int32),
          pltpu.VMEM((CHUNK,), jnp.int32),
          pltpu.VMEM((CHUNK, HD), jnp.float32),
          pltpu.VMEM((CHUNK, HD), jnp.float32),
          pltpu.VMEM((CHUNK, HD), jnp.float32),
          pltpu.VMEM((CHUNK, HD), jnp.float32),
          pltpu.VMEM((CHUNK, HD), jnp.float32),
          pltpu.VMEM((CROWS, HD), jnp.float32),
          pltpu.VMEM_SHARED((ACC_ROWS, HD), jnp.float32),
          pltpu.VMEM_SHARED((ACC_ROWS, HD), jnp.float32),
          pltpu.VMEM_SHARED((CROWS, HD), jnp.float32),
          pltpu.SemaphoreType.DMA,
          pltpu.SemaphoreType.DMA,
      ],
  )
  def agg(xa_hbm, xb_hbm, src_hbm, dst_hbm, suma_hbm, sumb_hbm, counts_hbm,
          src0, dst0, src1, dst1, ld_v, cr_v, ra0, rb0, ra1, rb1,
          onesbuf, ccopy, acca, accb, cacc, sem0, sem1):
    rowa, rowb = ra0, rb0
    c = lax.axis_index("c")
    s = lax.axis_index("s")
    wid = s * NC + c
    out0 = c * QMAX
    lo = base + out0
    hi = jnp.minimum(lo + QMAX, base + QHALF)
    lane = jnp.arange(L, dtype=jnp.int32)
    col0 = jnp.zeros((L,), jnp.int32)
    ones16 = jnp.ones((L,), jnp.float32)
    zeros16 = jnp.zeros((L,), jnp.float32)

    def zero_row(i, _):
      for j in range(HD // L):
        rowa[i, pl.ds(j * L, L)] = zeros16
        onesbuf[i, pl.ds(j * L, L)] = zeros16
      return 0
    lax.fori_loop(0, CHUNK, zero_row, 0)

    nzch = ACC_ROWS // ZCHUNK
    nz = (nzch - s + NS - 1) // NS

    def zero_acc(i, _):
      r0 = (s + i * NS) * ZCHUNK
      pltpu.sync_copy(rowa.at[pl.ds(0, ZCHUNK)], acca.at[pl.ds(r0, ZCHUNK)])
      pltpu.sync_copy(rowa.at[pl.ds(0, ZCHUNK)], accb.at[pl.ds(r0, ZCHUNK)])
      return 0
    lax.fori_loop(0, nz, zero_acc, 0)

    @pl.when(s == 0)
    def _zero_cacc():
      pltpu.sync_copy(onesbuf.at[pl.ds(0, CROWS)], cacc)

    plsc.subcore_barrier()

    nj = (NCHUNKS - s + NS - 1) // NS

    def issue(k, src_r, dst_r, ra, rb, sem):
      off = (s + k * NS) * CHUNK
      pltpu.sync_copy(src_hbm.at[pl.ds(off, CHUNK)], src_r)
      pltpu.sync_copy(dst_hbm.at[pl.ds(off, CHUNK)], dst_r)
      pltpu.async_copy(xa_hbm.at[src_r], ra, sem)
      pltpu.async_copy(xb_hbm.at[src_r], rb, sem)

    def process(src_r, dst_r, ra, rb, sem):
      for t in range(CHUNK // L):
        dv = dst_r[pl.ds(t * L, L)]
        in_range = (dv >= lo) & (dv < hi)
        ld = jnp.where(in_range, dv - lo, TRASH)
        ld_v[pl.ds(t * L, L)] = ld
        cr_v[pl.ds(t * L, L)] = lax.shift_right_logical(ld, 7)
        plsc.store_scatter(onesbuf, [lane + t * L, lax.bitwise_and(ld, 127)],
                           ones16)
      pltpu.make_async_copy(xa_hbm.at[src_r], ra, sem).wait()
      pltpu.sync_copy(ra, acca.at[ld_v], add=True)
      pltpu.make_async_copy(xb_hbm.at[src_r], rb, sem).wait()
      pltpu.sync_copy(rb, accb.at[ld_v], add=True)
      pltpu.sync_copy(onesbuf, cacc.at[cr_v], add=True)
      for t in range(CHUNK // L):
        ld = ld_v[pl.ds(t * L, L)]
        plsc.store_scatter(onesbuf, [lane + t * L, lax.bitwise_and(ld, 127)],
                           zeros16)

    issue(0, src0, dst0, ra0, rb0, sem0)
    npairs = nj // 2

    def pair(i, _):
      issue(2 * i + 1, src1, dst1, ra1, rb1, sem1)
      process(src0, dst0, ra0, rb0, sem0)

      @pl.when(2 * i + 2 < nj)
      def _prefetch():
        issue(2 * i + 2, src0, dst0, ra0, rb0, sem0)

      process(src1, dst1, ra1, rb1, sem1)
      return 0
    lax.fori_loop(0, npairs, pair, 0)

    @pl.when(2 * npairs < nj)
    def _odd_tail():
      process(src0, dst0, ra0, rb0, sem0)

    plsc.subcore_barrier()

    pltpu.sync_copy(cacc, ccopy)

    def emit_counts(n0, ngroups):
      for t in range(ngroups):
        vals = ccopy[n0 // HD, pl.ds(t * L, L)]
        plsc.store_scatter(rowa, [lane + t * L, col0], vals)

    nwb = (WB_FULL - s + NS - 1) // NS

    def wb_body(i, _):
      r0 = (s + i * NS) * WB
      pltpu.sync_copy(acca.at[pl.ds(r0, WB)], rowa)
      pltpu.sync_copy(rowa, suma_hbm.at[pl.ds(out0 + r0, WB)])
      pltpu.sync_copy(accb.at[pl.ds(r0, WB)], rowb)
      pltpu.sync_copy(rowb, sumb_hbm.at[pl.ds(out0 + r0, WB)])
      return 0
    lax.fori_loop(0, nwb, wb_body, 0)

    def wbc_body(i, _):
      r0 = (s + i * NS) * WB
      emit_counts(r0, WB // L)
      pltpu.sync_copy(rowa, counts_hbm.at[pl.ds(out0 + r0, WB)])
      return 0
    lax.fori_loop(0, nwb, wbc_body, 0)

    @pl.when(s == NS - 1)
    def _tail():
      r0 = WB_FULL * WB
      pltpu.sync_copy(acca.at[pl.ds(r0, WB_TAIL)], rowa.at[pl.ds(0, WB_TAIL)])
      pltpu.sync_copy(rowa.at[pl.ds(0, WB_TAIL)],
                      suma_hbm.at[pl.ds(out0 + r0, WB_TAIL)])
      pltpu.sync_copy(accb.at[pl.ds(r0, WB_TAIL)], rowb.at[pl.ds(0, WB_TAIL)])
      pltpu.sync_copy(rowb.at[pl.ds(0, WB_TAIL)],
                      sumb_hbm.at[pl.ds(out0 + r0, WB_TAIL)])
      emit_counts(r0, (WB_TAIL + L - 1) // L)
      pltpu.sync_copy(rowa.at[pl.ds(0, WB_TAIL)],
                      counts_hbm.at[pl.ds(out0 + r0, WB_TAIL)])

  return agg


BLK = 400


def _tc_body(x_ref, suma_ref, sumb_ref, counts_ref, wl_ref, wr_ref, b_ref,
             o_ref):
  cnt = jnp.maximum(counts_ref[:, 0:1], 1.0)
  meana = suma_ref[...] / cnt
  meanb = sumb_ref[...] / cnt
  o_ref[...] = (
      jnp.dot(meana, wl_ref[0:HD, :], preferred_element_type=jnp.float32,
              precision=lax.Precision.HIGHEST)
      + jnp.dot(meanb, wl_ref[HD:D, :], preferred_element_type=jnp.float32,
                precision=lax.Precision.HIGHEST)
      + jnp.dot(x_ref[...], wr_ref[...], preferred_element_type=jnp.float32,
                precision=lax.Precision.HIGHEST)
      + b_ref[...])


def _tc_update(x, suma, sumb, counts, W_l, W_r, b):
  return pl.pallas_call(
      _tc_body,
      grid=(N // BLK,),
      in_specs=[
          pl.BlockSpec((BLK, D), lambda i: (i, 0)),
          pl.BlockSpec((BLK, HD), lambda i: (i, 0)),
          pl.BlockSpec((BLK, HD), lambda i: (i, 0)),
          pl.BlockSpec((BLK, HD), lambda i: (i, 0)),
          pl.BlockSpec((D, D), lambda i: (0, 0)),
          pl.BlockSpec((D, D), lambda i: (0, 0)),
          pl.BlockSpec((1, D), lambda i: (0, 0)),
      ],
      out_specs=pl.BlockSpec((BLK, D), lambda i: (i, 0)),
      out_shape=jax.ShapeDtypeStruct((N, D), jnp.float32),
  )(x, suma, sumb, counts, W_l, W_r, b.reshape(1, D))


@jax.jit
def kernel(x, edge_index, W_l, W_r, b):
  src = edge_index[0]
  dst = edge_index[1]
  xa = x[:, :HD]
  xb = x[:, HD:]
  suma0, sumb0, counts0 = _make_sc_aggregate(0)(xa, xb, src, dst)
  suma1, sumb1, counts1 = _make_sc_aggregate(QHALF)(xa, xb, src, dst)
  suma = jnp.concatenate([suma0[:QHALF], suma1[:QHALF]])
  sumb = jnp.concatenate([sumb0[:QHALF], sumb1[:QHALF]])
  counts = jnp.concatenate([counts0[:QHALF], counts1[:QHALF]])
  return _tc_update(x, suma, sumb, counts, W_l, W_r, b)


# speedup vs baseline: 2.8462x; 1.6354x over previous
import functools

import jax
import jax.numpy as jnp
from jax import lax
from jax.experimental import pallas as pl
from jax.experimental.pallas import tpu as pltpu
from jax.experimental.pallas import tpu_sc as plsc

N, E, D = 10000, 160000, 256

NC, NS, L = 2, 16, 16
NW = NC * NS
HD = D // 2
QHALF = 5000
QMAX = 2504
NQ = 4
ACC_ROWS = 2528
TRASH = QMAX
CROWS = 20
OUT_ROWS = 5008
CHUNK = 128
NCHUNKS = E // CHUNK
LCAP = 5248
ZCHUNK = 32
WB = 128
WB_FULL = 19
WB_TAIL = 72

_MESH = plsc.VectorSubcoreMesh(
    core_axis_name="c", subcore_axis_name="s", num_cores=NC, num_subcores=NS)
_SC_PARAMS = pltpu.CompilerParams(needs_layout_passes=False)


@functools.partial(
    pl.kernel,
    out_type=(
        jax.ShapeDtypeStruct((NW * NQ * LCAP,), jnp.int32),
        jax.ShapeDtypeStruct((NW * NQ * LCAP,), jnp.int32),
        jax.ShapeDtypeStruct((NW * L,), jnp.int32),
    ),
    mesh=_MESH,
    compiler_params=_SC_PARAMS,
    scratch_types=[
        pltpu.VMEM((CHUNK,), jnp.int32),
        pltpu.VMEM((CHUNK,), jnp.int32),
        pltpu.VMEM((LCAP,), jnp.int32),
        pltpu.VMEM((LCAP,), jnp.int32),
        pltpu.VMEM((LCAP,), jnp.int32),
        pltpu.VMEM((LCAP,), jnp.int32),
        pltpu.VMEM((LCAP,), jnp.int32),
        pltpu.VMEM((LCAP,), jnp.int32),
        pltpu.VMEM((LCAP,), jnp.int32),
        pltpu.VMEM((LCAP,), jnp.int32),
        pltpu.VMEM((L,), jnp.int32),
    ],
)
def _sc_bucketize(src_hbm, dst_hbm, slist_hbm, llist_hbm, ccnt_hbm,
                  src_v, dst_v, sq0, sq1, sq2, sq3, lq0, lq1, lq2, lq3, cbuf):
  sqs = [sq0, sq1, sq2, sq3]
  lqs = [lq0, lq1, lq2, lq3]
  c = lax.axis_index("c")
  s = lax.axis_index("s")
  w = s * NC + c
  nk = (NCHUNKS - w + NW - 1) // NW
  trash16 = jnp.full((L,), TRASH, jnp.int32)
  zero16 = jnp.zeros((L,), jnp.int32)

  def chunk_body(i, offs):
    off_e = (w + i * NW) * CHUNK
    pltpu.sync_copy(src_hbm.at[pl.ds(off_e, CHUNK)], src_v)
    pltpu.sync_copy(dst_hbm.at[pl.ds(off_e, CHUNK)], dst_v)
    offs = list(offs)
    for t in range(CHUNK // L):
      sv = src_v[pl.ds(t * L, L)]
      dv = dst_v[pl.ds(t * L, L)]
      q = ((dv >= 2504).astype(jnp.int32) + (dv >= 5000).astype(jnp.int32)
           + (dv >= 7504).astype(jnp.int32))
      sub = (jnp.where(q == 1, 2504, 0) + jnp.where(q == 2, 5000, 0)
             + jnp.where(q == 3, 7504, 0))
      ld = dv - sub
      for qq in range(NQ):
        m = q == qq
        plsc.store_compressed(sqs[qq].at[pl.ds(offs[qq], L)], sv, mask=m)
        plsc.store_compressed(lqs[qq].at[pl.ds(offs[qq], L)], ld, mask=m)
        offs[qq] = offs[qq] + jnp.sum(m.astype(jnp.int32))
    return tuple(offs)

  zero = jnp.zeros((), jnp.int32)
  offs = lax.fori_loop(0, nk, chunk_body, (zero, zero, zero, zero))

  lane = jnp.arange(L, dtype=jnp.int32)
  cvec = jnp.zeros((L,), jnp.int32)
  for qq in range(NQ):
    off = offs[qq]
    for r in range(CHUNK // L):
      sqs[qq][pl.ds(off + r * L, L)] = zero16
      lqs[qq][pl.ds(off + r * L, L)] = trash16
    cvec = jnp.where(lane == qq, (off + CHUNK - 1) // CHUNK, cvec)
    pltpu.sync_copy(sqs[qq], slist_hbm.at[pl.ds((w * NQ + qq) * LCAP, LCAP)])
    pltpu.sync_copy(lqs[qq], llist_hbm.at[pl.ds((w * NQ + qq) * LCAP, LCAP)])
  cbuf[...] = cvec
  pltpu.sync_copy(cbuf, ccnt_hbm.at[pl.ds(w * L, L)])


def _make_sc_aggregate(base):
  qb = 0 if base == 0 else 2

  @functools.partial(
      pl.kernel,
      out_type=(
          jax.ShapeDtypeStruct((OUT_ROWS, HD), jnp.float32),
          jax.ShapeDtypeStruct((OUT_ROWS, HD), jnp.float32),
          jax.ShapeDtypeStruct((OUT_ROWS, HD), jnp.float32),
      ),
      mesh=_MESH,
      compiler_params=_SC_PARAMS,
      scratch_types=[
          pltpu.VMEM((CHUNK,), jnp.int32),
          pltpu.VMEM((CHUNK,), jnp.int32),
          pltpu.VMEM((CHUNK,), jnp.int32),
          pltpu.VMEM((CHUNK,), jnp.int32),
          pltpu.VMEM((CHUNK,), jnp.int32),
          pltpu.VMEM((NW * L,), jnp.int32),
          pltpu.VMEM((CHUNK, HD), jnp.float32),
          pltpu.VMEM((CHUNK, HD), jnp.float32),
          pltpu.VMEM((CHUNK, HD), jnp.float32),
          pltpu.VMEM((CHUNK, HD), jnp.float32),
          pltpu.VMEM((CHUNK, HD), jnp.float32),
          pltpu.VMEM((CROWS, HD), jnp.float32),
          pltpu.VMEM_SHARED((ACC_ROWS, HD), jnp.float32),
          pltpu.VMEM_SHARED((ACC_ROWS, HD), jnp.float32),
          pltpu.VMEM_SHARED((CROWS, HD), jnp.float32),
          pltpu.SemaphoreType.DMA,
          pltpu.SemaphoreType.DMA,
      ],
  )
  def agg(xa_hbm, xb_hbm, slist_hbm, llist_hbm, ccnt_hbm,
          suma_hbm, sumb_hbm, counts_hbm,
          src0, ldb0, src1, ldb1, cr_v, cv, ra0, rb0, ra1, rb1,
          onesbuf, ccopy, acca, accb, cacc, sem0, sem1):
    c = lax.axis_index("c")
    s = lax.axis_index("s")
    out0 = c * QMAX
    q_id = qb + c
    lane = jnp.arange(L, dtype=jnp.int32)
    col0 = jnp.zeros((L,), jnp.int32)
    ones16 = jnp.ones((L,), jnp.float32)
    zeros16 = jnp.zeros((L,), jnp.float32)

    def zero_row(i, _):
      for j in range(HD // L):
        ra0[i, pl.ds(j * L, L)] = zeros16
        onesbuf[i, pl.ds(j * L, L)] = zeros16
      return 0
    lax.fori_loop(0, CHUNK, zero_row, 0)

    nzch = ACC_ROWS // ZCHUNK
    nz = (nzch - s + NS - 1) // NS

    def zero_acc(i, _):
      r0 = (s + i * NS) * ZCHUNK
      pltpu.sync_copy(ra0.at[pl.ds(0, ZCHUNK)], acca.at[pl.ds(r0, ZCHUNK)])
      pltpu.sync_copy(ra0.at[pl.ds(0, ZCHUNK)], accb.at[pl.ds(r0, ZCHUNK)])
      return 0
    lax.fori_loop(0, nz, zero_acc, 0)

    @pl.when(s == 0)
    def _zero_cacc():
      pltpu.sync_copy(onesbuf.at[pl.ds(0, CROWS)], cacc)

    plsc.subcore_barrier()

    pltpu.sync_copy(ccnt_hbm, cv)
    w0 = 2 * s
    w1 = 2 * s + 1
    nv0 = cv[pl.ds(w0 * L, L)]
    nv1 = cv[pl.ds(w1 * L, L)]
    n0 = jnp.sum(jnp.where(lane == q_id, nv0, 0))
    n1 = jnp.sum(jnp.where(lane == q_id, nv1, 0))
    ntot = n0 + n1
    base0 = (w0 * NQ + q_id) * LCAP
    base1 = (w1 * NQ + q_id) * LCAP

    def off_of(k):
      return jnp.where(k < n0, base0 + k * CHUNK, base1 + (k - n0) * CHUNK)

    def issue(k, src_r, ld_r, ra, rb, sem):
      off = off_of(k)
      pltpu.sync_copy(slist_hbm.at[pl.ds(off, CHUNK)], src_r)
      pltpu.sync_copy(llist_hbm.at[pl.ds(off, CHUNK)], ld_r)
      pltpu.async_copy(xa_hbm.at[src_r], ra, sem)
      pltpu.async_copy(xb_hbm.at[src_r], rb, sem)

    def process(src_r, ld_r, ra, rb, sem):
      for t in range(CHUNK // L):
        ld = ld_r[pl.ds(t * L, L)]
        cr_v[pl.ds(t * L, L)] = lax.shift_right_logical(ld, 7)
        plsc.store_scatter(onesbuf, [lane + t * L, lax.bitwise_and(ld, 127)],
                           ones16)
      pltpu.make_async_copy(xa_hbm.at[src_r], ra, sem).wait()
      pltpu.sync_copy(ra, acca.at[ld_r], add=True)
      pltpu.make_async_copy(xb_hbm.at[src_r], rb, sem).wait()
      pltpu.sync_copy(rb, accb.at[ld_r], add=True)
      pltpu.sync_copy(onesbuf, cacc.at[cr_v], add=True)
      for t in range(CHUNK // L):
        ld = ld_r[pl.ds(t * L, L)]
        plsc.store_scatter(onesbuf, [lane + t * L, lax.bitwise_and(ld, 127)],
                           zeros16)

    @pl.when(ntot > 0)
    def _prologue():
      issue(0, src0, ldb0, ra0, rb0, sem0)

    npairs = ntot // 2

    def pair(i, _):
      issue(2 * i + 1, src1, ldb1, ra1, rb1, sem1)
      process(src0, ldb0, ra0, rb0, sem0)

      @pl.when(2 * i + 2 < ntot)
      def _prefetch():
        issue(2 * i + 2, src0, ldb0, ra0, rb0, sem0)

      process(src1, ldb1, ra1, rb1, sem1)
      return 0
    lax.fori_loop(0, npairs, pair, 0)

    @pl.when(2 * npairs < ntot)
    def _odd_tail():
      process(src0, ldb0, ra0, rb0, sem0)

    plsc.subcore_barrier()

    pltpu.sync_copy(cacc, ccopy)

    def emit_counts(r0_, ngroups):
      for t in range(ngroups):
        vals = ccopy[r0_ // HD, pl.ds(t * L, L)]
        plsc.store_scatter(ra0, [lane + t * L, col0], vals)

    nwb = (WB_FULL - s + NS - 1) // NS

    def wb_body(i, _):
      r0 = (s + i * NS) * WB
      pltpu.sync_copy(acca.at[pl.ds(r0, WB)], ra0)
      pltpu.sync_copy(ra0, suma_hbm.at[pl.ds(out0 + r0, WB)])
      pltpu.sync_copy(accb.at[pl.ds(r0, WB)], rb0)
      pltpu.sync_copy(rb0, sumb_hbm.at[pl.ds(out0 + r0, WB)])
      return 0
    lax.fori_loop(0, nwb, wb_body, 0)

    def wbc_body(i, _):
      r0 = (s + i * NS) * WB
      emit_counts(r0, WB // L)
      pltpu.sync_copy(ra0, counts_hbm.at[pl.ds(out0 + r0, WB)])
      return 0
    lax.fori_loop(0, nwb, wbc_body, 0)

    @pl.when(s == NS - 1)
    def _tail():
      r0 = WB_FULL * WB
      pltpu.sync_copy(acca.at[pl.ds(r0, WB_TAIL)], ra0.at[pl.ds(0, WB_TAIL)])
      pltpu.sync_copy(ra0.at[pl.ds(0, WB_TAIL)],
                      suma_hbm.at[pl.ds(out0 + r0, WB_TAIL)])
      pltpu.sync_copy(accb.at[pl.ds(r0, WB_TAIL)], rb0.at[pl.ds(0, WB_TAIL)])
      pltpu.sync_copy(rb0.at[pl.ds(0, WB_TAIL)],
                      sumb_hbm.at[pl.ds(out0 + r0, WB_TAIL)])
      emit_counts(r0, (WB_TAIL + L - 1) // L)
      pltpu.sync_copy(ra0.at[pl.ds(0, WB_TAIL)],
                      counts_hbm.at[pl.ds(out0 + r0, WB_TAIL)])

  return agg


BLK = 400


def _tc_body(x_ref, suma_ref, sumb_ref, counts_ref, wl_ref, wr_ref, b_ref,
             o_ref):
  cnt = jnp.maximum(counts_ref[:, 0:1], 1.0)
  meana = suma_ref[...] / cnt
  meanb = sumb_ref[...] / cnt
  o_ref[...] = (
      jnp.dot(meana, wl_ref[0:HD, :], preferred_element_type=jnp.float32,
              precision=lax.Precision.HIGHEST)
      + jnp.dot(meanb, wl_ref[HD:D, :], preferred_element_type=jnp.float32,
                precision=lax.Precision.HIGHEST)
      + jnp.dot(x_ref[...], wr_ref[...], preferred_element_type=jnp.float32,
                precision=lax.Precision.HIGHEST)
      + b_ref[...])


def _tc_update(x, suma, sumb, counts, W_l, W_r, b):
  return pl.pallas_call(
      _tc_body,
      grid=(N // BLK,),
      in_specs=[
          pl.BlockSpec((BLK, D), lambda i: (i, 0)),
          pl.BlockSpec((BLK, HD), lambda i: (i, 0)),
          pl.BlockSpec((BLK, HD), lambda i: (i, 0)),
          pl.BlockSpec((BLK, HD), lambda i: (i, 0)),
          pl.BlockSpec((D, D), lambda i: (0, 0)),
          pl.BlockSpec((D, D), lambda i: (0, 0)),
          pl.BlockSpec((1, D), lambda i: (0, 0)),
      ],
      out_specs=pl.BlockSpec((BLK, D), lambda i: (i, 0)),
      out_shape=jax.ShapeDtypeStruct((N, D), jnp.float32),
  )(x, suma, sumb, counts, W_l, W_r, b.reshape(1, D))


@jax.jit
def kernel(x, edge_index, W_l, W_r, b):
  src = edge_index[0]
  dst = edge_index[1]
  xa = x[:, :HD]
  xb = x[:, HD:]
  slist, llist, ccnt = _sc_bucketize(src, dst)
  suma0, sumb0, counts0 = _make_sc_aggregate(0)(xa, xb, slist, llist, ccnt)
  suma1, sumb1, counts1 = _make_sc_aggregate(QHALF)(xa, xb, slist, llist, ccnt)
  suma = jnp.concatenate([suma0[:QHALF], suma1[:QHALF]])
  sumb = jnp.concatenate([sumb0[:QHALF], sumb1[:QHALF]])
  counts = jnp.concatenate([counts0[:QHALF], counts1[:QHALF]])
  return _tc_update(x, suma, sumb, counts, W_l, W_r, b)
